# baseline (device time: 22198 ns/iter reference)
import functools

import jax
import jax.numpy as jnp
from jax import lax
from jax.experimental import pallas as pl
from jax.experimental.pallas import tpu as pltpu

M = 1024
N = 512


def kernel(x, dest):
    dest2d = dest.reshape(1, M)

    def body(x_ref, dest_ref, out_ref, xsend_ref, xpeer_ref, dpeer_ref,
             send_sems, recv_sems):
        my_x = lax.axis_index("x")
        my_y = lax.axis_index("y")
        peer = (1 - my_x, my_y)

        xsend_ref[...] = x_ref[...].astype(jnp.bfloat16)

        barrier_sem = pltpu.get_barrier_semaphore()
        pl.semaphore_signal(barrier_sem, inc=1, device_id=peer,
                            device_id_type=pl.DeviceIdType.MESH)
        pl.semaphore_wait(barrier_sem, 1)

        rdma_d = pltpu.make_async_remote_copy(
            src_ref=dest_ref, dst_ref=dpeer_ref,
            send_sem=send_sems.at[1], recv_sem=recv_sems.at[1],
            device_id=peer, device_id_type=pl.DeviceIdType.MESH)
        rdma_d.start()
        rdma_x = pltpu.make_async_remote_copy(
            src_ref=xsend_ref, dst_ref=xpeer_ref,
            send_sem=send_sems.at[0], recv_sem=recv_sems.at[0],
            device_id=peer, device_id_type=pl.DeviceIdType.MESH)
        rdma_x.start()
        rdma_d.wait()

        dl = dest_ref[...]
        dp = dpeer_ref[...]
        ml = (dl == my_x)
        mp = (dp == my_x)
        iota_i = lax.broadcasted_iota(jnp.int32, (M, M), 0)
        iota_j = lax.broadcasted_iota(jnp.int32, (M, M), 1)
        tri = (iota_i <= iota_j).astype(jnp.float32)
        csl = jnp.dot(ml.astype(jnp.float32), tri,
                      preferred_element_type=jnp.float32).astype(jnp.int32)
        csp = jnp.dot(mp.astype(jnp.float32), tri,
                      preferred_element_type=jnp.float32).astype(jnp.int32)
        cl = csl[0, M - 1]
        cp = csp[0, M - 1]
        off_l = jnp.where(my_x == 0, 0, cp)
        off_p = jnp.where(my_x == 0, cl, 0)
        posl = off_l + csl - 1
        posp = off_p + csp - 1

        p_l = ((iota_i == posl) & ml).astype(jnp.bfloat16)
        p_p = ((iota_i == posp) & mp).astype(jnp.bfloat16)
        acc = jnp.dot(p_l, xsend_ref[...], preferred_element_type=jnp.float32)

        rdma_x.wait()
        acc = acc + jnp.dot(p_p, xpeer_ref[...],
                            preferred_element_type=jnp.float32)
        out_ref[...] = acc.astype(jnp.bfloat16)

        @functools.partial(pl.run_scoped, sem2=pltpu.SemaphoreType.REGULAR)
        def _(sem2):
            pl.semaphore_signal(sem2, inc=1, device_id=peer,
                                device_id_type=pl.DeviceIdType.MESH)
            pl.semaphore_wait(sem2, 1)

    return pl.pallas_call(
        body,
        out_shape=jax.ShapeDtypeStruct((M, N), jnp.bfloat16),
        in_specs=[pl.BlockSpec(memory_space=pltpu.VMEM),
                  pl.BlockSpec(memory_space=pltpu.VMEM)],
        out_specs=pl.BlockSpec(memory_space=pltpu.VMEM),
        scratch_shapes=[
            pltpu.VMEM((M, N), jnp.bfloat16),
            pltpu.VMEM((M, N), jnp.bfloat16),
            pltpu.VMEM((1, M), jnp.int32),
            pltpu.SemaphoreType.DMA((2,)),
            pltpu.SemaphoreType.DMA((2,)),
        ],
        compiler_params=pltpu.CompilerParams(collective_id=0),
    )(x, dest2d)


# device time: 18831 ns/iter; 1.1788x vs baseline; 1.1788x over previous
import functools

import jax
import jax.numpy as jnp
from jax import lax
from jax.experimental import pallas as pl
from jax.experimental.pallas import tpu as pltpu

M = 1024
N = 512
H = M // 2
CH = 128
NCH = H // CH


def kernel(x, dest):
    dest2d = dest.reshape(1, M)

    def body(x_ref, dest_ref, out_ref, xsend_ref, xbuf_ref, ybuf_ref,
             dpeer_ref, dsems, xs_send, xs_recv, ys_send, ys_recv):
        my_x = lax.axis_index("x")
        my_y = lax.axis_index("y")
        xpeer = (1 - my_x, my_y)
        ypeer = (my_x, 1 - my_y)

        xsend_ref[...] = x_ref[...].astype(jnp.bfloat16)

        barrier_sem = pltpu.get_barrier_semaphore()
        for nbr in (xpeer, ypeer):
            pl.semaphore_signal(barrier_sem, inc=1, device_id=nbr,
                                device_id_type=pl.DeviceIdType.MESH)
        pl.semaphore_wait(barrier_sem, 2)

        rdma_d = pltpu.make_async_remote_copy(
            src_ref=dest_ref, dst_ref=dpeer_ref,
            send_sem=dsems.at[0], recv_sem=dsems.at[1],
            device_id=xpeer, device_id_type=pl.DeviceIdType.MESH)
        rdma_d.start()

        half0 = my_y * H
        x_rdmas = []
        for k in range(NCH):
            r = pltpu.make_async_remote_copy(
                src_ref=xsend_ref.at[pl.ds(half0 + k * CH, CH)],
                dst_ref=xbuf_ref.at[pl.ds(k * CH, CH)],
                send_sem=xs_send.at[k], recv_sem=xs_recv.at[k],
                device_id=xpeer, device_id_type=pl.DeviceIdType.MESH)
            r.start()
            x_rdmas.append(r)

        rdma_d.wait()

        dl = dest_ref[...]
        dp = dpeer_ref[...]
        ml = (dl == my_x)
        mp = (dp == my_x)
        iota_i = lax.broadcasted_iota(jnp.int32, (M, M), 0)
        iota_j = lax.broadcasted_iota(jnp.int32, (M, M), 1)
        tri = (iota_i <= iota_j).astype(jnp.float32)
        csl = jnp.dot(ml.astype(jnp.float32), tri,
                      preferred_element_type=jnp.float32).astype(jnp.int32)
        csp = jnp.dot(mp.astype(jnp.float32), tri,
                      preferred_element_type=jnp.float32).astype(jnp.int32)
        cl = csl[0, M - 1]
        cp = csp[0, M - 1]
        off_l = jnp.where(my_x == 0, 0, cp)
        off_p = jnp.where(my_x == 0, cl, 0)
        posl = off_l + csl - 1
        posp = off_p + csp - 1

        def arrival_order(v):
            swapped = jnp.concatenate([v[:, H:], v[:, :H]], axis=1)
            return jnp.where(my_y == 0, v, swapped)

        posp_a = arrival_order(posp)
        mp_a = arrival_order(mp.astype(jnp.int32))
        p_l = ((iota_i == posl) & ml).astype(jnp.bfloat16)
        p_p = ((iota_i == posp_a) & (mp_a > 0)).astype(jnp.bfloat16)
        acc = jnp.dot(p_l, xsend_ref[...], preferred_element_type=jnp.float32)

        y_rdmas = []
        for k in range(NCH):
            x_rdmas[k].wait_recv()
            r = pltpu.make_async_remote_copy(
                src_ref=xbuf_ref.at[pl.ds(k * CH, CH)],
                dst_ref=ybuf_ref.at[pl.ds(k * CH, CH)],
                send_sem=ys_send.at[k], recv_sem=ys_recv.at[k],
                device_id=ypeer, device_id_type=pl.DeviceIdType.MESH)
            r.start()
            y_rdmas.append(r)
        for k in range(NCH):
            acc = acc + jnp.dot(p_p[:, k * CH:(k + 1) * CH],
                                xbuf_ref[pl.ds(k * CH, CH), :],
                                preferred_element_type=jnp.float32)
        for k in range(NCH):
            y_rdmas[k].wait_recv()
            acc = acc + jnp.dot(p_p[:, H + k * CH:H + (k + 1) * CH],
                                ybuf_ref[pl.ds(k * CH, CH), :],
                                preferred_element_type=jnp.float32)

        out_ref[...] = acc.astype(jnp.bfloat16)

        for k in range(NCH):
            x_rdmas[k].wait_send()
            y_rdmas[k].wait_send()

        @functools.partial(pl.run_scoped, sem2=pltpu.SemaphoreType.REGULAR)
        def _(sem2):
            for nbr in (xpeer, ypeer):
                pl.semaphore_signal(sem2, inc=1, device_id=nbr,
                                    device_id_type=pl.DeviceIdType.MESH)
            pl.semaphore_wait(sem2, 2)

    return pl.pallas_call(
        body,
        out_shape=jax.ShapeDtypeStruct((M, N), jnp.bfloat16),
        in_specs=[pl.BlockSpec(memory_space=pltpu.VMEM),
                  pl.BlockSpec(memory_space=pltpu.VMEM)],
        out_specs=pl.BlockSpec(memory_space=pltpu.VMEM),
        scratch_shapes=[
            pltpu.VMEM((M, N), jnp.bfloat16),
            pltpu.VMEM((H, N), jnp.bfloat16),
            pltpu.VMEM((H, N), jnp.bfloat16),
            pltpu.VMEM((1, M), jnp.int32),
            pltpu.SemaphoreType.DMA((2,)),
            pltpu.SemaphoreType.DMA((NCH,)),
            pltpu.SemaphoreType.DMA((NCH,)),
            pltpu.SemaphoreType.DMA((NCH,)),
            pltpu.SemaphoreType.DMA((NCH,)),
        ],
        compiler_params=pltpu.CompilerParams(collective_id=0),
    )(x, dest2d)


# device time: 18334 ns/iter; 1.2108x vs baseline; 1.0271x over previous
import functools

import jax
import jax.numpy as jnp
from jax import lax
from jax.experimental import pallas as pl
from jax.experimental.pallas import tpu as pltpu

M = 1024
N = 512
H = M // 2
CH = 128
NCH = H // CH


def kernel(x, dest):
    dest2d = dest.reshape(1, M)

    def body(x_ref, dest_ref, out_ref, xsend_ref, xbuf_ref, ybuf_ref,
             dpeer_ref, dsems, xs_send, xs_recv, ys_send, ys_recv):
        my_x = lax.axis_index("x")
        my_y = lax.axis_index("y")
        xpeer = (1 - my_x, my_y)
        ypeer = (my_x, 1 - my_y)

        xsend_ref[...] = x_ref[...].astype(jnp.bfloat16)

        barrier_sem = pltpu.get_barrier_semaphore()
        for nbr in (xpeer, ypeer):
            pl.semaphore_signal(barrier_sem, inc=1, device_id=nbr,
                                device_id_type=pl.DeviceIdType.MESH)
        pl.semaphore_wait(barrier_sem, 2)

        rdma_d = pltpu.make_async_remote_copy(
            src_ref=dest_ref, dst_ref=dpeer_ref,
            send_sem=dsems.at[0], recv_sem=dsems.at[1],
            device_id=xpeer, device_id_type=pl.DeviceIdType.MESH)
        rdma_d.start()

        half0 = my_y * H
        x_rdmas = []
        for k in range(NCH):
            r = pltpu.make_async_remote_copy(
                src_ref=xsend_ref.at[pl.ds(half0 + k * CH, CH)],
                dst_ref=xbuf_ref.at[pl.ds(k * CH, CH)],
                send_sem=xs_send.at[k], recv_sem=xs_recv.at[k],
                device_id=xpeer, device_id_type=pl.DeviceIdType.MESH)
            r.start()
            x_rdmas.append(r)

        iota_i = lax.broadcasted_iota(jnp.int32, (M, M), 0)
        iota_j = lax.broadcasted_iota(jnp.int32, (M, M), 1)
        tri = (iota_i <= iota_j).astype(jnp.float32)

        rdma_d.wait()

        dl = dest_ref[...]
        dp = dpeer_ref[...]
        ml = (dl == my_x)
        mp = (dp == my_x)
        csl = jnp.dot(ml.astype(jnp.float32), tri,
                      preferred_element_type=jnp.float32).astype(jnp.int32)
        csp = jnp.dot(mp.astype(jnp.float32), tri,
                      preferred_element_type=jnp.float32).astype(jnp.int32)
        cl = csl[0, M - 1]
        cp = csp[0, M - 1]
        off_l = jnp.where(my_x == 0, 0, cp)
        off_p = jnp.where(my_x == 0, cl, 0)
        posl = off_l + csl - 1
        posp = off_p + csp - 1

        def arrival_order(v):
            swapped = jnp.concatenate([v[:, H:], v[:, :H]], axis=1)
            return jnp.where(my_y == 0, v, swapped)

        posp_a = arrival_order(posp)
        mp_a = arrival_order(mp.astype(jnp.int32))

        def fwd(k):
            x_rdmas[k].wait_recv()
            r = pltpu.make_async_remote_copy(
                src_ref=xbuf_ref.at[pl.ds(k * CH, CH)],
                dst_ref=ybuf_ref.at[pl.ds(k * CH, CH)],
                send_sem=ys_send.at[k], recv_sem=ys_recv.at[k],
                device_id=ypeer, device_id_type=pl.DeviceIdType.MESH)
            r.start()
            y_rdmas.append(r)

        y_rdmas = []
        fwd(0)
        p_l = ((iota_i == posl) & ml).astype(jnp.bfloat16)
        fwd(1)
        p_p = ((iota_i == posp_a) & (mp_a > 0)).astype(jnp.bfloat16)
        fwd(2)
        acc = jnp.dot(p_l, xsend_ref[...], preferred_element_type=jnp.float32)
        fwd(3)
        for k in range(NCH):
            acc = acc + jnp.dot(p_p[:, k * CH:(k + 1) * CH],
                                xbuf_ref[pl.ds(k * CH, CH), :],
                                preferred_element_type=jnp.float32)
        for k in range(NCH):
            y_rdmas[k].wait_recv()
            acc = acc + jnp.dot(p_p[:, H + k * CH:H + (k + 1) * CH],
                                ybuf_ref[pl.ds(k * CH, CH), :],
                                preferred_element_type=jnp.float32)

        out_ref[...] = acc.astype(jnp.bfloat16)

        for k in range(NCH):
            x_rdmas[k].wait_send()
            y_rdmas[k].wait_send()

        @functools.partial(pl.run_scoped, sem2=pltpu.SemaphoreType.REGULAR)
        def _(sem2):
            for nbr in (xpeer, ypeer):
                pl.semaphore_signal(sem2, inc=1, device_id=nbr,
                                    device_id_type=pl.DeviceIdType.MESH)
            pl.semaphore_wait(sem2, 2)

    return pl.pallas_call(
        body,
        out_shape=jax.ShapeDtypeStruct((M, N), jnp.bfloat16),
        in_specs=[pl.BlockSpec(memory_space=pltpu.VMEM),
                  pl.BlockSpec(memory_space=pltpu.VMEM)],
        out_specs=pl.BlockSpec(memory_space=pltpu.VMEM),
        scratch_shapes=[
            pltpu.VMEM((M, N), jnp.bfloat16),
            pltpu.VMEM((H, N), jnp.bfloat16),
            pltpu.VMEM((H, N), jnp.bfloat16),
            pltpu.VMEM((1, M), jnp.int32),
            pltpu.SemaphoreType.DMA((2,)),
            pltpu.SemaphoreType.DMA((NCH,)),
            pltpu.SemaphoreType.DMA((NCH,)),
            pltpu.SemaphoreType.DMA((NCH,)),
            pltpu.SemaphoreType.DMA((NCH,)),
        ],
        compiler_params=pltpu.CompilerParams(collective_id=0),
    )(x, dest2d)
